# speedup vs baseline: 1.1385x; 1.1385x over previous
import functools

import jax
import jax.numpy as jnp
from jax import lax
from jax.experimental import pallas as pl
from jax.experimental.pallas import tpu as pltpu
from jax.experimental.pallas import tpu_sc as plsc

VOCAB = 100000
D = 128
W = 5
B = 16384

NC = 2
NS = 16
NW = NC * NS
BPW = B // NW
CH = 128
NK = BPW // CH
LANES = 16
GROUPS = D // LANES


def _mesh():
    return plsc.VectorSubcoreMesh(core_axis_name="c", subcore_axis_name="s")


@functools.partial(
    pl.kernel,
    out_type=jax.ShapeDtypeStruct((B, D), jnp.float32),
    mesh=_mesh(),
    scratch_types=[
        pltpu.VMEM((W, CH), jnp.int32),
        pltpu.VMEM((W, CH, D), jnp.float32),
        pltpu.VMEM((W, D), jnp.float32),
        pltpu.VMEM((D,), jnp.float32),
        pltpu.SemaphoreType.DMA,
    ],
)
def _word_encode(word_hbm, table_hbm, pos_hbm, out_hbm,
                 idx_v, rows_v, pos_v, possum_v, sem):
    wid = lax.axis_index("s") * NC + lax.axis_index("c")
    base = wid * BPW

    pltpu.sync_copy(pos_hbm, pos_v)
    for c in range(GROUPS):
        sl = pl.ds(c * LANES, LANES)
        s = pos_v[0, sl]
        for i in range(1, W):
            s = s + pos_v[i, sl]
        possum_v[sl] = s

    for k in range(NK):
        col0 = base + k * CH
        pltpu.sync_copy(word_hbm.at[:, pl.ds(col0, CH)], idx_v)
        descs = [
            pltpu.async_copy(table_hbm.at[idx_v.at[i]], rows_v.at[i], sem)
            for i in range(W)
        ]
        for d in descs:
            d.wait()

        def body(r, _):
            for c in range(GROUPS):
                sl = pl.ds(c * LANES, LANES)
                s = rows_v[0, r, sl]
                for i in range(1, W):
                    s = s + rows_v[i, r, sl]
                rows_v[0, r, sl] = s + possum_v[sl]
            return 0

        lax.fori_loop(0, CH, body, 0)

        pltpu.sync_copy(rows_v.at[0], out_hbm.at[pl.ds(col0, CH), :])


def kernel(word, letter_table, pos_table):
    word = word.astype(jnp.int32)
    return _word_encode(word, letter_table, pos_table)


# speedup vs baseline: 1.5913x; 1.3978x over previous
import functools

import jax
import jax.numpy as jnp
from jax import lax
from jax.experimental import pallas as pl
from jax.experimental.pallas import tpu as pltpu
from jax.experimental.pallas import tpu_sc as plsc

VOCAB = 100000
D = 128
W = 5
B = 16384

NC = 2
NS = 16
NW = NC * NS
BPW = B // NW
CH = 64
NK = BPW // CH
NBUF = 2
LANES = 16
GROUPS = D // LANES


def _mesh():
    return plsc.VectorSubcoreMesh(core_axis_name="c", subcore_axis_name="s")


@functools.partial(
    pl.kernel,
    out_type=jax.ShapeDtypeStruct((B, D), jnp.float32),
    mesh=_mesh(),
    scratch_types=[
        pltpu.VMEM((NBUF, W, 2 * CH), jnp.int32),
        pltpu.VMEM((NBUF, W, CH, D), jnp.float32),
        pltpu.VMEM((W, D), jnp.float32),
        pltpu.VMEM((D,), jnp.float32),
        pltpu.SemaphoreType.DMA,
        pltpu.SemaphoreType.DMA,
    ],
)
def _word_encode(word_hbm, table_hbm, pos_hbm, out_hbm,
                 idx_v, rows_v, pos_v, possum_v, gsem, osem):
    wid = lax.axis_index("s") * NC + lax.axis_index("c")
    base = wid * BPW

    pltpu.sync_copy(pos_hbm, pos_v)
    for c in range(GROUPS):
        sl = pl.ds(c * LANES, LANES)
        s = pos_v[0, sl]
        for i in range(1, W):
            s = s + pos_v[i, sl]
        possum_v[sl] = s

    def fire(k):
        bf = k % NBUF
        p = k // 2
        half = (k % 2) * CH
        if k % 2 == 0:
            pltpu.sync_copy(word_hbm.at[:, pl.ds(base + p * 2 * CH, 2 * CH)],
                            idx_v.at[p % 2])
        return [
            pltpu.async_copy(table_hbm.at[idx_v.at[p % 2, i, pl.ds(half, CH)]],
                             rows_v.at[bf, i], gsem)
            for i in range(W)
        ]

    gather_descs = [None] * NK
    out_descs = [None] * NK
    gather_descs[0] = fire(0)

    carry0 = tuple(possum_v[pl.ds(c * LANES, LANES)] for c in range(GROUPS))

    for k in range(NK):
        bf = k % NBUF
        if k + 1 < NK:
            if k + 1 - NBUF >= 0 and out_descs[k + 1 - NBUF] is not None:
                out_descs[k + 1 - NBUF].wait()
            gather_descs[k + 1] = fire(k + 1)
        for dsc in gather_descs[k]:
            dsc.wait()

        def body(r, ps):
            for c in range(GROUPS):
                sl = pl.ds(c * LANES, LANES)
                a = rows_v[bf, 0, r, sl] + rows_v[bf, 1, r, sl]
                b2 = rows_v[bf, 2, r, sl] + rows_v[bf, 3, r, sl]
                e = (a + b2) + (rows_v[bf, 4, r, sl] + ps[c])
                rows_v[bf, 0, r, sl] = e
            return ps

        lax.fori_loop(0, CH, body, carry0)

        col0 = base + k * CH
        out_descs[k] = pltpu.async_copy(
            rows_v.at[bf, 0], out_hbm.at[pl.ds(col0, CH), :], osem)

    for k in range(NK - NBUF, NK):
        if out_descs[k] is not None:
            out_descs[k].wait()


def kernel(word, letter_table, pos_table):
    word = word.astype(jnp.int32)
    return _word_encode(word, letter_table, pos_table)


# speedup vs baseline: 1.5939x; 1.0016x over previous
import functools

import jax
import jax.numpy as jnp
from jax import lax
from jax.experimental import pallas as pl
from jax.experimental.pallas import tpu as pltpu
from jax.experimental.pallas import tpu_sc as plsc

VOCAB = 100000
D = 128
W = 5
B = 16384

NC = 2
NS = 16
NW = NC * NS
BPW = B // NW
CH = 128
NBUF = BPW // CH
LANES = 16
GROUPS = D // LANES


def _mesh():
    return plsc.VectorSubcoreMesh(core_axis_name="c", subcore_axis_name="s")


@functools.partial(
    pl.kernel,
    out_type=jax.ShapeDtypeStruct((B, D), jnp.float32),
    mesh=_mesh(),
    scratch_types=[
        pltpu.VMEM((NBUF, W, CH), jnp.int32),
        pltpu.VMEM((NBUF, CH, D), jnp.float32),
        pltpu.VMEM((W, D), jnp.float32),
        pltpu.SemaphoreType.DMA,
        pltpu.SemaphoreType.DMA,
    ],
)
def _word_encode(word_hbm, table_hbm, pos_hbm, out_hbm,
                 idx_v, acc_v, pos_v, gsem, osem):
    wid = lax.axis_index("s") * NC + lax.axis_index("c")
    base = wid * BPW

    pltpu.sync_copy(pos_hbm, pos_v)
    possum = []
    for c in range(GROUPS):
        sl = pl.ds(c * LANES, LANES)
        s = (pos_v[0, sl] + pos_v[1, sl]) + (pos_v[2, sl] + pos_v[3, sl])
        possum.append(s + pos_v[4, sl])

    for j in range(NBUF):
        pltpu.sync_copy(word_hbm.at[:, pl.ds(base + j * CH, CH)], idx_v.at[j])

    def build(r, ps):
        for j in range(NBUF):
            for c in range(GROUPS):
                acc_v[j, r, pl.ds(c * LANES, LANES)] = ps[c]
        return ps

    lax.fori_loop(0, CH, build, tuple(possum))

    for r in range(W):
        descs = [
            pltpu.async_copy(table_hbm.at[idx_v.at[j, r]], acc_v.at[j],
                             gsem, add=True)
            for j in range(NBUF)
        ]
        for dsc in descs:
            dsc.wait()

    outs = [
        pltpu.async_copy(acc_v.at[j], out_hbm.at[pl.ds(base + j * CH, CH), :],
                         osem)
        for j in range(NBUF)
    ]
    for dsc in outs:
        dsc.wait()


def kernel(word, letter_table, pos_table):
    word = word.astype(jnp.int32)
    return _word_encode(word, letter_table, pos_table)


# speedup vs baseline: 1.6976x; 1.0651x over previous
import functools

import jax
import jax.numpy as jnp
from jax import lax
from jax.experimental import pallas as pl
from jax.experimental.pallas import tpu as pltpu
from jax.experimental.pallas import tpu_sc as plsc

VOCAB = 100000
D = 128
W = 5
B = 16384

NC = 2
NS = 16
NW = NC * NS
BPW = B // NW
CH = 128
NBUF = BPW // CH
LANES = 16
GROUPS = D // LANES


def _mesh():
    return plsc.VectorSubcoreMesh(core_axis_name="c", subcore_axis_name="s")


@functools.partial(
    pl.kernel,
    out_type=jax.ShapeDtypeStruct((B, D), jnp.float32),
    mesh=_mesh(),
    scratch_types=[
        pltpu.VMEM((NBUF, W, CH), jnp.int32),
        pltpu.VMEM((NBUF, CH, D), jnp.float32),
        pltpu.VMEM((W, D), jnp.float32),
        pltpu.SemaphoreType.DMA,
        pltpu.SemaphoreType.DMA,
        pltpu.SemaphoreType.DMA,
        pltpu.SemaphoreType.DMA,
        pltpu.SemaphoreType.DMA,
        pltpu.SemaphoreType.DMA,
    ],
)
def _word_encode(word_hbm, table_hbm, pos_hbm, out_hbm,
                 idx_v, acc_v, pos_v, isem, g0, g1, g2, g3, osem):
    wid = lax.axis_index("s") * NC + lax.axis_index("c")
    base = wid * BPW
    gsem = [g0, g1, g2, g3]

    idx_descs = [
        pltpu.async_copy(word_hbm.at[:, pl.ds(base + j * CH, CH)],
                         idx_v.at[j], isem)
        for j in range(NBUF)
    ]

    pltpu.sync_copy(pos_hbm, pos_v)
    possum = []
    for c in range(GROUPS):
        sl = pl.ds(c * LANES, LANES)
        s = (pos_v[0, sl] + pos_v[1, sl]) + (pos_v[2, sl] + pos_v[3, sl])
        possum.append(s + pos_v[4, sl])

    def build(r, ps):
        for j in range(NBUF):
            for c in range(GROUPS):
                acc_v[j, r, pl.ds(c * LANES, LANES)] = ps[c]
        return ps

    lax.fori_loop(0, CH, build, tuple(possum))
    for dsc in idx_descs:
        dsc.wait()

    descs = {}
    for r in range(W):
        for j in range(NBUF):
            if r > 0:
                descs[(j, r - 1)].wait()
            descs[(j, r)] = pltpu.async_copy(
                table_hbm.at[idx_v.at[j, r]], acc_v.at[j], gsem[j], add=True)

    outs = []
    for j in range(NBUF):
        descs[(j, W - 1)].wait()
        outs.append(pltpu.async_copy(
            acc_v.at[j], out_hbm.at[pl.ds(base + j * CH, CH), :], osem))
    for dsc in outs:
        dsc.wait()


def kernel(word, letter_table, pos_table):
    word = word.astype(jnp.int32)
    return _word_encode(word, letter_table, pos_table)


# speedup vs baseline: 1.8499x; 1.0897x over previous
import functools

import jax
import jax.numpy as jnp
from jax import lax
from jax.experimental import pallas as pl
from jax.experimental.pallas import tpu as pltpu
from jax.experimental.pallas import tpu_sc as plsc

VOCAB = 100000
D = 128
W = 5
B = 16384

NC = 2
NS = 16
NW = NC * NS
BPW = B // NW
CH = 64
NBUF = BPW // CH
NPAIR = NBUF // 2
LANES = 16
GROUPS = D // LANES


def _mesh():
    return plsc.VectorSubcoreMesh(core_axis_name="c", subcore_axis_name="s")


@functools.partial(
    pl.kernel,
    out_type=jax.ShapeDtypeStruct((B, D), jnp.float32),
    mesh=_mesh(),
    scratch_types=[
        pltpu.VMEM((NPAIR, W, 2 * CH), jnp.int32),
        pltpu.VMEM((NBUF, CH, D), jnp.float32),
        pltpu.VMEM((W, D), jnp.float32),
        pltpu.SemaphoreType.DMA,
        pltpu.SemaphoreType.DMA,
        pltpu.SemaphoreType.DMA,
        pltpu.SemaphoreType.DMA,
        pltpu.SemaphoreType.DMA,
        pltpu.SemaphoreType.DMA,
        pltpu.SemaphoreType.DMA,
        pltpu.SemaphoreType.DMA,
        pltpu.SemaphoreType.DMA,
        pltpu.SemaphoreType.DMA,
    ],
)
def _word_encode(word_hbm, table_hbm, pos_hbm, out_hbm,
                 idx_v, acc_v, pos_v, isem,
                 g0, g1, g2, g3, g4, g5, g6, g7, osem):
    wid = lax.axis_index("s") * NC + lax.axis_index("c")
    base = wid * BPW
    gsem = [g0, g1, g2, g3, g4, g5, g6, g7]

    idx_descs = [
        pltpu.async_copy(word_hbm.at[:, pl.ds(base + p * 2 * CH, 2 * CH)],
                         idx_v.at[p], isem)
        for p in range(NPAIR)
    ]

    pltpu.sync_copy(pos_hbm, pos_v)
    possum = []
    for c in range(GROUPS):
        sl = pl.ds(c * LANES, LANES)
        s = (pos_v[0, sl] + pos_v[1, sl]) + (pos_v[2, sl] + pos_v[3, sl])
        possum.append(s + pos_v[4, sl])

    for dsc in idx_descs:
        dsc.wait()

    def idx_slice(j, r):
        return idx_v.at[j // 2, r, pl.ds((j % 2) * CH, CH)]

    descs = {}
    for j in range(NBUF):
        def build(r, ps, j=j):
            for c in range(GROUPS):
                acc_v[j, r, pl.ds(c * LANES, LANES)] = ps[c]
            return ps

        lax.fori_loop(0, CH, build, tuple(possum))
        descs[(j, 0)] = pltpu.async_copy(
            table_hbm.at[idx_slice(j, 0)], acc_v.at[j], gsem[j], add=True)

    for r in range(1, W):
        for j in range(NBUF):
            descs[(j, r - 1)].wait()
            descs[(j, r)] = pltpu.async_copy(
                table_hbm.at[idx_slice(j, r)], acc_v.at[j], gsem[j], add=True)

    outs = []
    for j in range(NBUF):
        descs[(j, W - 1)].wait()
        outs.append(pltpu.async_copy(
            acc_v.at[j], out_hbm.at[pl.ds(base + j * CH, CH), :], osem))
    for dsc in outs:
        dsc.wait()


def kernel(word, letter_table, pos_table):
    word = word.astype(jnp.int32)
    return _word_encode(word, letter_table, pos_table)
